# baseline (speedup 1.0000x reference)
import dataclasses
import functools

import jax
import jax.numpy as jnp
from jax import lax
from jax.experimental import pallas as pl
from jax.experimental.pallas import tpu as pltpu
from jax.experimental.pallas import tpu_sc as plsc

ROWS = 16384
COLS = 1000
K = 200

NUM_CORES = 2
NUM_SUBCORES = 16
NW = NUM_CORES * NUM_SUBCORES
ROWS_PER_WORKER = ROWS // NW

R = 32
NCHUNK = ROWS_PER_WORKER // R

_OFFS = tuple(range(0, K - 16, 16)) + (K - 16,)

_mesh = plsc.VectorSubcoreMesh(core_axis_name="c", subcore_axis_name="s")

_cp = pltpu.CompilerParams(skip_device_barrier=True)
if "needs_layout_passes" in pltpu.CompilerParams.__dataclass_fields__:
    _cp = dataclasses.replace(_cp, needs_layout_passes=False)


@functools.partial(
    pl.kernel,
    mesh=_mesh,
    compiler_params=_cp,
    out_type=jax.ShapeDtypeStruct((ROWS, K), jnp.float32),
    scratch_types=[
        pltpu.VMEM((R, COLS), jnp.float32),
        pltpu.VMEM((R, K), jnp.int32),
        pltpu.VMEM((R, K), jnp.float32),
        pltpu.VMEM((R, COLS), jnp.float32),
        pltpu.VMEM((R, K), jnp.int32),
        pltpu.VMEM((R, K), jnp.float32),
        pltpu.SemaphoreType.DMA,
        pltpu.SemaphoreType.DMA,
        pltpu.SemaphoreType.DMA,
        pltpu.SemaphoreType.DMA,
    ],
)
def _gather_dim1(
    in_hbm, idx_hbm, out_hbm,
    in0, idx0, out0, in1, idx1, out1,
    sl0, sl1, so0, so1,
):
    wid = lax.axis_index("s") * NUM_CORES + lax.axis_index("c")
    row0 = wid * ROWS_PER_WORKER
    bufs = ((in0, idx0, out0, sl0, so0), (in1, idx1, out1, sl1, so1))

    def load_descs(c, b):
        in_v, idx_v, _, sl, _ = bufs[b]
        base = row0 + c * R
        return (
            pltpu.make_async_copy(in_hbm.at[pl.ds(base, R)], in_v, sl),
            pltpu.make_async_copy(idx_hbm.at[pl.ds(base, R)], idx_v, sl),
        )

    def out_desc(c, b):
        _, _, out_v, _, so = bufs[b]
        base = row0 + c * R
        return pltpu.make_async_copy(out_v, out_hbm.at[pl.ds(base, R)], so)

    def compute(b):
        in_v, idx_v, out_v, _, _ = bufs[b]

        @pl.loop(0, R, step=4)
        def _(r):
            for rr in range(4):
                rsplat = jnp.full((16,), r + rr, jnp.int32)
                for off in _OFFS:
                    idx16 = idx_v[r + rr, pl.ds(off, 16)]
                    vals = plsc.load_gather(in_v, [rsplat, idx16])
                    out_v[r + rr, pl.ds(off, 16)] = vals

    for d in load_descs(0, 0):
        d.start()

    @pl.loop(0, NCHUNK // 2)
    def _(cc):
        for b in range(2):
            c = cc * 2 + b
            for d in load_descs(c, b):
                d.wait()

            @pl.when(c + 1 < NCHUNK)
            def _():
                for d in load_descs(c + 1, 1 - b):
                    d.start()

            @pl.when(c >= 2)
            def _():
                out_desc(c - 2, b).wait()

            compute(b)
            out_desc(c, b).start()

    out_desc(NCHUNK - 2, 0).wait()
    out_desc(NCHUNK - 1, 1).wait()


def kernel(input, index):
    return _gather_dim1(input, index.astype(jnp.int32))


# speedup vs baseline: 1.1036x; 1.1036x over previous
import dataclasses
import functools

import jax
import jax.numpy as jnp
from jax import lax
from jax.experimental import pallas as pl
from jax.experimental.pallas import tpu as pltpu
from jax.experimental.pallas import tpu_sc as plsc

ROWS = 16384
COLS = 1000
K = 200

NUM_CORES = 2
NUM_SUBCORES = 16
NW = NUM_CORES * NUM_SUBCORES
ROWS_PER_WORKER = ROWS // NW

R = 32
NCHUNK = ROWS_PER_WORKER // R

_OFFS = tuple(range(0, K - 16, 16)) + (K - 16,)

_mesh = plsc.VectorSubcoreMesh(core_axis_name="c", subcore_axis_name="s")

_cp = pltpu.CompilerParams(skip_device_barrier=True)
if "needs_layout_passes" in pltpu.CompilerParams.__dataclass_fields__:
    _cp = dataclasses.replace(_cp, needs_layout_passes=False)


@functools.partial(
    pl.kernel,
    mesh=_mesh,
    compiler_params=_cp,
    out_type=jax.ShapeDtypeStruct((ROWS, K), jnp.float32),
    scratch_types=[
        pltpu.VMEM((R, COLS), jnp.float32),
        pltpu.VMEM((R, K), jnp.int32),
        pltpu.VMEM((R, K), jnp.float32),
        pltpu.VMEM((R, COLS), jnp.float32),
        pltpu.VMEM((R, K), jnp.int32),
        pltpu.VMEM((R, K), jnp.float32),
        pltpu.SemaphoreType.DMA,
        pltpu.SemaphoreType.DMA,
        pltpu.SemaphoreType.DMA,
        pltpu.SemaphoreType.DMA,
    ],
)
def _gather_dim1(
    in_hbm, idx_hbm, out_hbm,
    in0, idx0, out0, in1, idx1, out1,
    sl0, sl1, so0, so1,
):
    wid = lax.axis_index("s") * NUM_CORES + lax.axis_index("c")
    row0 = wid * ROWS_PER_WORKER
    bufs = ((in0, idx0, out0, sl0, so0), (in1, idx1, out1, sl1, so1))

    def load_descs(c, b):
        in_v, idx_v, _, sl, _ = bufs[b]
        base = row0 + c * R
        return (
            pltpu.make_async_copy(in_hbm.at[pl.ds(base, R)], in_v, sl),
            pltpu.make_async_copy(idx_hbm.at[pl.ds(base, R)], idx_v, sl),
        )

    def out_desc(c, b):
        _, _, out_v, _, so = bufs[b]
        base = row0 + c * R
        return pltpu.make_async_copy(out_v, out_hbm.at[pl.ds(base, R)], so)

    def compute(b):
        in_v, idx_v, out_v, _, _ = bufs[b]

        @pl.loop(0, R, step=2)
        def _(r):
            for rr in range(2):
                rsplat = jnp.full((16,), r + rr, jnp.int32)
                for off in _OFFS:
                    idx16 = idx_v[r + rr, pl.ds(off, 16)]
                    vals = plsc.load_gather(in_v, [rsplat, idx16])
                    out_v[r + rr, pl.ds(off, 16)] = vals

    for d in load_descs(0, 0):
        d.start()

    @pl.loop(0, NCHUNK // 2)
    def _(cc):
        for b in range(2):
            c = cc * 2 + b
            for d in load_descs(c, b):
                d.wait()

            @pl.when(c + 1 < NCHUNK)
            def _():
                for d in load_descs(c + 1, 1 - b):
                    d.start()

            @pl.when(c >= 2)
            def _():
                out_desc(c - 2, b).wait()

            compute(b)
            out_desc(c, b).start()

    out_desc(NCHUNK - 2, 0).wait()
    out_desc(NCHUNK - 1, 1).wait()


def kernel(input, index):
    return _gather_dim1(input, index.astype(jnp.int32))
